# baseline (speedup 1.0000x reference)
import functools

import jax
import jax.numpy as jnp
import numpy as np
from jax import lax
from jax.experimental import pallas as pl
from jax.experimental.pallas import tpu as pltpu
from jax.experimental.pallas import tpu_sc as plsc

D_MODEL = 1024
MAX_LEN = 32768
SEG_DIM = 64
LANES = 16


def _sc_body(S, B, bnd_hbm, orig_hbm, ctab_hbm,
             start_hbm, end_hbm, fsum_hbm, bsum_hbm, cg_hbm,
             bnd_v, res_v, sum_v, idx_v, rows_v, sem):
    HALF = S // 2
    n_chunks = HALF // LANES
    cid = lax.axis_index("c")
    sid = lax.axis_index("s")
    wid = sid * 2 + cid
    dirn = wid // 16
    task = wid % 16
    row = task // 2
    half = task % 2
    base = half * HALF

    GW = (B * S) // 32
    wpr = S // GW
    gb = wid // wpr
    gs0 = (wid % wpr) * GW
    GC = 128
    for c in range(GW // GC):
        off = gs0 + c * GC
        pltpu.sync_copy(orig_hbm.at[gb, pl.ds(off, GC)], idx_v)
        pltpu.async_copy(ctab_hbm.at[idx_v], rows_v, sem).wait()
        pltpu.sync_copy(rows_v, cg_hbm.at[gb, pl.ds(off, GC)])

    pltpu.sync_copy(bnd_hbm.at[row, pl.ds(base, HALF)], bnd_v.at[pl.ds(0, HALF)])

    @pl.when(dirn == 0)
    def _fwd():
        def fbody(l, carry):
            idx = lax.iota(jnp.int32, LANES) + (base + l * LANES)
            b = bnd_v[pl.ds(l * LANES, LANES)]
            flag = (b != 0) | (idx == 0)
            v = jnp.where(flag, idx, -1)
            st = jnp.maximum(plsc.cummax(v), carry)
            res_v[pl.ds(l * LANES, LANES)] = st
            return jnp.max(st)

        fc = lax.fori_loop(0, n_chunks, fbody, jnp.int32(-1))
        pltpu.sync_copy(res_v, start_hbm.at[row, 0, pl.ds(base, HALF)])

        @pl.when(half == 0)
        def _():
            sum_v[...] = jnp.full((LANES,), fc, jnp.int32)
            pltpu.sync_copy(sum_v, fsum_hbm.at[row])

    @pl.when(dirn == 1)
    def _bwd():
        bnd_v[pl.ds(HALF, LANES)] = jnp.zeros((LANES,), jnp.int32)

        def bbody(k, carry):
            l = n_chunks - 1 - k
            off = l * LANES + 1
            u = bnd_v[pl.ds(off, LANES)]
            idx2 = lax.iota(jnp.int32, LANES) + (base + off)
            m = jnp.where(u != 0, idx2, S)
            r = lax.rev(m, (0,))
            ipm = -plsc.cummax(-r)
            esm = lax.rev(ipm, (0,))
            e = jnp.minimum(esm, carry)
            res_v[pl.ds(l * LANES, LANES)] = e
            return jnp.minimum(jnp.min(m), carry)

        bc = lax.fori_loop(0, n_chunks, bbody, jnp.int32(S))
        pltpu.sync_copy(res_v, end_hbm.at[row, 0, pl.ds(base, HALF)])

        @pl.when(half == 1)
        def _():
            b0 = bnd_v[pl.ds(0, LANES)]
            li = lax.iota(jnp.int32, LANES)
            m0 = jnp.where((li == 0) & (b0 != 0), jnp.int32(base), S)
            bs = jnp.minimum(bc, jnp.min(m0))
            sum_v[...] = jnp.full((LANES,), bs, jnp.int32)
            pltpu.sync_copy(sum_v, bsum_hbm.at[row])


def _sc_segments(bnd, orig, ctab):
    B, S = bnd.shape
    HALF = S // 2
    mesh = plsc.VectorSubcoreMesh(core_axis_name="c", subcore_axis_name="s")
    return pl.kernel(
        functools.partial(_sc_body, S, B),
        out_type=[
            jax.ShapeDtypeStruct((B, 1, S), jnp.int32),
            jax.ShapeDtypeStruct((B, 1, S), jnp.int32),
            jax.ShapeDtypeStruct((B, LANES), jnp.int32),
            jax.ShapeDtypeStruct((B, LANES), jnp.int32),
            jax.ShapeDtypeStruct((B, S, 128), jnp.float32),
        ],
        mesh=mesh,
        compiler_params=pltpu.CompilerParams(needs_layout_passes=False,
                                             use_tc_tiling_on_sc=True),
        scratch_types=[
            pltpu.VMEM((HALF + LANES,), jnp.int32),
            pltpu.VMEM((HALF,), jnp.int32),
            pltpu.VMEM((LANES,), jnp.int32),
            pltpu.VMEM((128,), jnp.int32),
            pltpu.VMEM((128, 128), jnp.float32),
            pltpu.SemaphoreType.DMA,
        ],
    )(bnd, orig, ctab)


def _assemble_body(S, S_blk, base_pe_ref, wproj_ref, wseg_ref, bseg_ref,
                   bproj_ref, ss_ref, se_ref, op_ref, cg_ref, mfb_ref, pm_ref,
                   fs_ref, bs_ref, out_ref, shared_sc, wc_sc):
    i = pl.program_id(0)
    j = pl.program_id(1)
    HALF = S // 2

    @pl.when(j == 0)
    def _():
        W1 = wproj_ref[:, :D_MODEL]
        W2 = wproj_ref[:, D_MODEL:]
        sh0 = lax.dot_general(base_pe_ref[...], W1, (((1,), (1,)), ((), ())),
                              preferred_element_type=jnp.float32)
        sb = lax.dot_general(bseg_ref[...], W2, (((1,), (1,)), ((), ())),
                             preferred_element_type=jnp.float32)
        shared_sc[...] = sh0 + bproj_ref[...] + sb

    @pl.when((i == 0) & (j == 0))
    def _():
        W2 = wproj_ref[:, D_MODEL:]
        wc_sc[...] = lax.dot_general(wseg_ref[...], W2, (((0,), (1,)), ((), ())),
                                     preferred_element_type=jnp.float32)

    sh = shared_sc[...]
    s_loc = ss_ref[0]
    e_loc = se_ref[0]
    pos_row = op_ref[0]
    fs = jnp.max(fs_ref[0])
    bs = jnp.min(bs_ref[0])
    first = i < (HALF // S_blk)
    s = jnp.maximum(s_loc, jnp.where(first, -1, fs))
    e = jnp.minimum(e_loc, jnp.where(first, bs, S))
    tok = lax.broadcasted_iota(jnp.int32, (1, S_blk), 1) + i * S_blk
    ln = (e - s).astype(jnp.float32)
    pos_in = (tok - s).astype(jnp.float32) / jnp.maximum(ln - 1.0, 1.0)
    len_norm = ln * (1.0 / float(S))
    gpos = pos_row.astype(jnp.float32) * (1.0 / float(MAX_LEN))
    fT = jnp.concatenate([pos_in, len_norm, gpos], axis=0)
    seg_add = lax.dot_general(fT, wc_sc[...], (((0,), (0,)), ((), ())),
                              preferred_element_type=jnp.float32)
    r = sh + seg_add
    posf_row = op_ref[0].astype(jnp.float32)
    motif_t = jnp.sin(mfb_ref[...] * posf_row)
    mg_add = lax.dot_general(motif_t, pm_ref[...], (((0,), (0,)), ((), ())),
                             preferred_element_type=jnp.float32)
    full = jnp.concatenate([
        r[:, 0:128] + cg_ref[0],
        r[:, 128:896],
        r[:, 896:1024] + mg_add,
    ], axis=1)
    out_ref[...] = full[None]


def _assemble_call(base_pe, W_proj, W_seg, b_seg2, b_proj2,
                   ss2, se2, op2, cg, mf, pmat, fsum3, bsum3):
    B = ss2.shape[0]
    S = base_pe.shape[0]
    S_blk = 512
    n_s = S // S_blk
    mfb = jnp.broadcast_to(mf[:, None], (16, S_blk))
    row_spec = pl.BlockSpec((1, 1, S_blk), lambda i, j: (j, 0, i))
    return pl.pallas_call(
        functools.partial(_assemble_body, S, S_blk),
        grid=(n_s, B),
        in_specs=[
            pl.BlockSpec((S_blk, D_MODEL), lambda i, j: (i, 0)),
            pl.BlockSpec((D_MODEL, D_MODEL + SEG_DIM), lambda i, j: (0, 0)),
            pl.BlockSpec((SEG_DIM, 3), lambda i, j: (0, 0)),
            pl.BlockSpec((1, SEG_DIM), lambda i, j: (0, 0)),
            pl.BlockSpec((1, D_MODEL), lambda i, j: (0, 0)),
            row_spec, row_spec, row_spec,
            pl.BlockSpec((1, S_blk, 128), lambda i, j: (j, i, 0)),
            pl.BlockSpec((16, S_blk), lambda i, j: (0, 0)),
            pl.BlockSpec((16, 128), lambda i, j: (0, 0)),
            pl.BlockSpec((1, 1, LANES), lambda i, j: (j, 0, 0)),
            pl.BlockSpec((1, 1, LANES), lambda i, j: (j, 0, 0)),
        ],
        out_specs=pl.BlockSpec((1, S_blk, D_MODEL), lambda i, j: (j, i, 0)),
        out_shape=jax.ShapeDtypeStruct((B, S, D_MODEL), jnp.float32),
        scratch_shapes=[
            pltpu.VMEM((S_blk, D_MODEL), jnp.float32),
            pltpu.VMEM((3, D_MODEL), jnp.float32),
        ],
    )(base_pe, W_proj, W_seg, b_seg2, b_proj2,
      ss2, se2, op2, cg, mfb, pmat, fsum3, bsum3)


@functools.lru_cache(maxsize=2)
def _base_pe_table(seq_len, d_model):
    pos = np.arange(seq_len, dtype=np.float32)[:, None]
    div = np.exp(np.arange(0, d_model, 2, dtype=np.float32)
                 * (-np.log(10000.0) / d_model)).astype(np.float32)
    pe = np.zeros((seq_len, d_model), dtype=np.float32)
    pe[:, 0::2] = np.sin((pos * div).astype(np.float32))
    pe[:, 1::2] = np.cos((pos * div).astype(np.float32))
    return jnp.asarray(pe)


@functools.lru_cache(maxsize=1)
def _sin_tables():
    cf = np.arange(0, D_MODEL // 4, 2, dtype=np.float32) * (2.0 * np.pi / 3.0)
    cf = cf.astype(np.float32)
    parts = []
    for period in [8, 10, 21, 147]:
        parts.append(np.arange(0, min(D_MODEL // 8, 8), 2, dtype=np.float32)
                     * (2.0 * np.pi / period))
    mf = np.concatenate(parts).astype(np.float32)
    p = np.arange(MAX_LEN, dtype=np.float32)[:, None]
    cargs = (p * cf[None, :]).astype(np.float32)
    ctab = (np.float32(0.1)
            * np.sin(cargs.astype(np.float64)).astype(np.float32))
    pmat = np.zeros((16, 128), dtype=np.float32)
    pmat[np.arange(16), 112 + np.arange(16)] = 0.05
    return jnp.asarray(ctab.astype(np.float32)), jnp.asarray(mf), jnp.asarray(pmat)


def kernel(seq_len, patch_boundaries, original_positions,
           W_seg, b_seg, W_proj, b_proj):
    B, S = patch_boundaries.shape
    bnd = patch_boundaries.astype(jnp.int32)
    orig = original_positions.astype(jnp.int32)

    ctab, mf, pmat = _sin_tables()
    seg_start, seg_end, fsum, bsum, cg = _sc_segments(bnd, orig, ctab)

    base_pe = _base_pe_table(S, D_MODEL)
    out = _assemble_call(
        base_pe, W_proj, W_seg,
        b_seg.reshape(1, SEG_DIM), b_proj.reshape(1, D_MODEL),
        seg_start, seg_end,
        orig.reshape(B, 1, S), cg, mf, pmat,
        fsum.reshape(B, 1, LANES), bsum.reshape(B, 1, LANES))
    return out


# speedup vs baseline: 1.0590x; 1.0590x over previous
import functools

import jax
import jax.numpy as jnp
import numpy as np
from jax import lax
from jax.experimental import pallas as pl
from jax.experimental.pallas import tpu as pltpu
from jax.experimental.pallas import tpu_sc as plsc

D_MODEL = 1024
MAX_LEN = 32768
SEG_DIM = 64
LANES = 16


def _sc_body(S, B, bnd_hbm, orig_hbm, ctab_hbm,
             start_hbm, end_hbm, fsum_hbm, bsum_hbm, cg_hbm,
             bnd_v, res_v, sum_v, idx_v, rows_v, sem):
    HALF = S // 2
    n_chunks = HALF // LANES
    cid = lax.axis_index("c")
    sid = lax.axis_index("s")
    wid = sid * 2 + cid
    dirn = wid // 16
    task = wid % 16
    row = task // 2
    half = task % 2
    base = half * HALF

    GW = (B * S) // 32
    wpr = S // GW
    gb = wid // wpr
    gs0 = (wid % wpr) * GW
    GC = 128
    NCH = GW // GC
    handles = []
    for c in range(NCH):
        off = gs0 + c * GC
        pltpu.sync_copy(orig_hbm.at[gb, pl.ds(off, GC)], idx_v.at[c])
        handles.append(pltpu.async_copy(ctab_hbm.at[idx_v.at[c]],
                                        rows_v.at[c], sem))

    pltpu.sync_copy(bnd_hbm.at[row, pl.ds(base, HALF)], bnd_v.at[pl.ds(0, HALF)])

    @pl.when(dirn == 0)
    def _fwd():
        def fbody(l, carry):
            idx = lax.iota(jnp.int32, LANES) + (base + l * LANES)
            b = bnd_v[pl.ds(l * LANES, LANES)]
            flag = (b != 0) | (idx == 0)
            v = jnp.where(flag, idx, -1)
            st = jnp.maximum(plsc.cummax(v), carry)
            res_v[pl.ds(l * LANES, LANES)] = st
            return jnp.max(st)

        fc = lax.fori_loop(0, n_chunks, fbody, jnp.int32(-1))
        pltpu.sync_copy(res_v, start_hbm.at[row, 0, pl.ds(base, HALF)])

        @pl.when(half == 0)
        def _():
            sum_v[...] = jnp.full((LANES,), fc, jnp.int32)
            pltpu.sync_copy(sum_v, fsum_hbm.at[row])

    @pl.when(dirn == 1)
    def _bwd():
        bnd_v[pl.ds(HALF, LANES)] = jnp.zeros((LANES,), jnp.int32)

        def bbody(k, carry):
            l = n_chunks - 1 - k
            off = l * LANES + 1
            u = bnd_v[pl.ds(off, LANES)]
            idx2 = lax.iota(jnp.int32, LANES) + (base + off)
            m = jnp.where(u != 0, idx2, S)
            r = lax.rev(m, (0,))
            ipm = -plsc.cummax(-r)
            esm = lax.rev(ipm, (0,))
            e = jnp.minimum(esm, carry)
            res_v[pl.ds(l * LANES, LANES)] = e
            return jnp.minimum(jnp.min(m), carry)

        bc = lax.fori_loop(0, n_chunks, bbody, jnp.int32(S))
        pltpu.sync_copy(res_v, end_hbm.at[row, 0, pl.ds(base, HALF)])

        @pl.when(half == 1)
        def _():
            b0 = bnd_v[pl.ds(0, LANES)]
            li = lax.iota(jnp.int32, LANES)
            m0 = jnp.where((li == 0) & (b0 != 0), jnp.int32(base), S)
            bs = jnp.minimum(bc, jnp.min(m0))
            sum_v[...] = jnp.full((LANES,), bs, jnp.int32)
            pltpu.sync_copy(sum_v, bsum_hbm.at[row])

    for c in range(NCH):
        handles[c].wait()
        pltpu.sync_copy(rows_v.at[c], cg_hbm.at[gb, pl.ds(gs0 + c * GC, GC)])


def _sc_segments(bnd, orig, ctab):
    B, S = bnd.shape
    HALF = S // 2
    mesh = plsc.VectorSubcoreMesh(core_axis_name="c", subcore_axis_name="s")
    return pl.kernel(
        functools.partial(_sc_body, S, B),
        out_type=[
            jax.ShapeDtypeStruct((B, 1, S), jnp.int32),
            jax.ShapeDtypeStruct((B, 1, S), jnp.int32),
            jax.ShapeDtypeStruct((B, LANES), jnp.int32),
            jax.ShapeDtypeStruct((B, LANES), jnp.int32),
            jax.ShapeDtypeStruct((B, S, 128), jnp.float32),
        ],
        mesh=mesh,
        compiler_params=pltpu.CompilerParams(needs_layout_passes=False,
                                             use_tc_tiling_on_sc=True),
        scratch_types=[
            pltpu.VMEM((HALF + LANES,), jnp.int32),
            pltpu.VMEM((HALF,), jnp.int32),
            pltpu.VMEM((LANES,), jnp.int32),
            pltpu.VMEM((4, 128), jnp.int32),
            pltpu.VMEM((4, 128, 128), jnp.float32),
            pltpu.SemaphoreType.DMA,
        ],
    )(bnd, orig, ctab)


def _assemble_body(S, S_blk, base_pe_ref, wproj_ref, wseg_ref, bseg_ref,
                   bproj_ref, ss_ref, se_ref, op_ref, cg_ref, mfb_ref, pm_ref,
                   fs_ref, bs_ref, out_ref, shared_sc, wc_sc):
    i = pl.program_id(0)
    j = pl.program_id(1)
    HALF = S // 2

    @pl.when(j == 0)
    def _():
        W1 = wproj_ref[:, :D_MODEL]
        W2 = wproj_ref[:, D_MODEL:]
        sh0 = lax.dot_general(base_pe_ref[...], W1, (((1,), (1,)), ((), ())),
                              preferred_element_type=jnp.float32)
        sb = lax.dot_general(bseg_ref[...], W2, (((1,), (1,)), ((), ())),
                             preferred_element_type=jnp.float32)
        shared_sc[...] = sh0 + bproj_ref[...] + sb

    @pl.when((i == 0) & (j == 0))
    def _():
        W2 = wproj_ref[:, D_MODEL:]
        wc_sc[...] = lax.dot_general(wseg_ref[...], W2, (((0,), (1,)), ((), ())),
                                     preferred_element_type=jnp.float32)

    sh = shared_sc[...]
    s_loc = ss_ref[0]
    e_loc = se_ref[0]
    pos_row = op_ref[0]
    fs = jnp.max(fs_ref[0])
    bs = jnp.min(bs_ref[0])
    first = i < (HALF // S_blk)
    s = jnp.maximum(s_loc, jnp.where(first, -1, fs))
    e = jnp.minimum(e_loc, jnp.where(first, bs, S))
    tok = lax.broadcasted_iota(jnp.int32, (1, S_blk), 1) + i * S_blk
    ln = (e - s).astype(jnp.float32)
    pos_in = (tok - s).astype(jnp.float32) / jnp.maximum(ln - 1.0, 1.0)
    len_norm = ln * (1.0 / float(S))
    gpos = pos_row.astype(jnp.float32) * (1.0 / float(MAX_LEN))
    fT = jnp.concatenate([pos_in, len_norm, gpos], axis=0)
    seg_add = lax.dot_general(fT, wc_sc[...], (((0,), (0,)), ((), ())),
                              preferred_element_type=jnp.float32)
    r = sh + seg_add
    posf_row = op_ref[0].astype(jnp.float32)
    motif_t = jnp.sin(mfb_ref[...] * posf_row)
    mg_add = lax.dot_general(motif_t, pm_ref[...], (((0,), (0,)), ((), ())),
                             preferred_element_type=jnp.float32)
    full = jnp.concatenate([
        r[:, 0:128] + cg_ref[0],
        r[:, 128:896],
        r[:, 896:1024] + mg_add,
    ], axis=1)
    out_ref[...] = full[None]


def _assemble_call(base_pe, W_proj, W_seg, b_seg2, b_proj2,
                   ss2, se2, op2, cg, mf, pmat, fsum3, bsum3):
    B = ss2.shape[0]
    S = base_pe.shape[0]
    S_blk = 512
    n_s = S // S_blk
    mfb = jnp.broadcast_to(mf[:, None], (16, S_blk))
    row_spec = pl.BlockSpec((1, 1, S_blk), lambda i, j: (j, 0, i))
    return pl.pallas_call(
        functools.partial(_assemble_body, S, S_blk),
        grid=(n_s, B),
        in_specs=[
            pl.BlockSpec((S_blk, D_MODEL), lambda i, j: (i, 0)),
            pl.BlockSpec((D_MODEL, D_MODEL + SEG_DIM), lambda i, j: (0, 0)),
            pl.BlockSpec((SEG_DIM, 3), lambda i, j: (0, 0)),
            pl.BlockSpec((1, SEG_DIM), lambda i, j: (0, 0)),
            pl.BlockSpec((1, D_MODEL), lambda i, j: (0, 0)),
            row_spec, row_spec, row_spec,
            pl.BlockSpec((1, S_blk, 128), lambda i, j: (j, i, 0)),
            pl.BlockSpec((16, S_blk), lambda i, j: (0, 0)),
            pl.BlockSpec((16, 128), lambda i, j: (0, 0)),
            pl.BlockSpec((1, 1, LANES), lambda i, j: (j, 0, 0)),
            pl.BlockSpec((1, 1, LANES), lambda i, j: (j, 0, 0)),
        ],
        out_specs=pl.BlockSpec((1, S_blk, D_MODEL), lambda i, j: (j, i, 0)),
        out_shape=jax.ShapeDtypeStruct((B, S, D_MODEL), jnp.float32),
        scratch_shapes=[
            pltpu.VMEM((S_blk, D_MODEL), jnp.float32),
            pltpu.VMEM((3, D_MODEL), jnp.float32),
        ],
    )(base_pe, W_proj, W_seg, b_seg2, b_proj2,
      ss2, se2, op2, cg, mfb, pmat, fsum3, bsum3)


@functools.lru_cache(maxsize=2)
def _base_pe_table(seq_len, d_model):
    pos = np.arange(seq_len, dtype=np.float32)[:, None]
    div = np.exp(np.arange(0, d_model, 2, dtype=np.float32)
                 * (-np.log(10000.0) / d_model)).astype(np.float32)
    pe = np.zeros((seq_len, d_model), dtype=np.float32)
    pe[:, 0::2] = np.sin((pos * div).astype(np.float32))
    pe[:, 1::2] = np.cos((pos * div).astype(np.float32))
    return jnp.asarray(pe)


@functools.lru_cache(maxsize=1)
def _sin_tables():
    cf = np.arange(0, D_MODEL // 4, 2, dtype=np.float32) * (2.0 * np.pi / 3.0)
    cf = cf.astype(np.float32)
    parts = []
    for period in [8, 10, 21, 147]:
        parts.append(np.arange(0, min(D_MODEL // 8, 8), 2, dtype=np.float32)
                     * (2.0 * np.pi / period))
    mf = np.concatenate(parts).astype(np.float32)
    p = np.arange(MAX_LEN, dtype=np.float32)[:, None]
    cargs = (p * cf[None, :]).astype(np.float32)
    ctab = (np.float32(0.1)
            * np.sin(cargs.astype(np.float64)).astype(np.float32))
    ctab = jnp.asarray(ctab)
    pmat = np.zeros((16, 128), dtype=np.float32)
    pmat[np.arange(16), 112 + np.arange(16)] = 0.05
    return ctab, jnp.asarray(mf), jnp.asarray(pmat)


def kernel(seq_len, patch_boundaries, original_positions,
           W_seg, b_seg, W_proj, b_proj):
    B, S = patch_boundaries.shape
    bnd = patch_boundaries.astype(jnp.int32)
    orig = original_positions.astype(jnp.int32)

    ctab, mf, pmat = _sin_tables()
    seg_start, seg_end, fsum, bsum, cg = _sc_segments(bnd, orig, ctab)

    base_pe = _base_pe_table(S, D_MODEL)
    out = _assemble_call(
        base_pe, W_proj, W_seg,
        b_seg.reshape(1, SEG_DIM), b_proj.reshape(1, D_MODEL),
        seg_start, seg_end,
        orig.reshape(B, 1, S), cg, mf, pmat,
        fsum.reshape(B, 1, LANES), bsum.reshape(B, 1, LANES))
    return out


# speedup vs baseline: 1.2127x; 1.1452x over previous
import functools

import jax
import jax.numpy as jnp
import numpy as np
from jax import lax
from jax.experimental import pallas as pl
from jax.experimental.pallas import tpu as pltpu
from jax.experimental.pallas import tpu_sc as plsc

D_MODEL = 1024
MAX_LEN = 32768
SEG_DIM = 64
LANES = 16


def _sc_body(S, B, bnd_hbm, orig_hbm, ctab_hbm,
             start_hbm, end_hbm, fsum_hbm, bsum_hbm, cg_hbm,
             bnd_v, res_v, sum_v, idx_v, rows_v, sem):
    HALF = S // 2
    n_chunks = HALF // LANES
    cid = lax.axis_index("c")
    sid = lax.axis_index("s")
    wid = sid * 2 + cid
    dirn = wid // 16
    task = wid % 16
    row = task // 2
    half = task % 2
    base = half * HALF

    GW = (B * S) // 32
    wpr = S // GW
    gb = wid // wpr
    gs0 = (wid % wpr) * GW
    GC = 128
    NCH = GW // GC
    handles = []
    for c in range(NCH):
        off = gs0 + c * GC
        pltpu.sync_copy(orig_hbm.at[gb, pl.ds(off, GC)], idx_v.at[c])
        handles.append(pltpu.async_copy(ctab_hbm.at[idx_v.at[c]],
                                        rows_v.at[c], sem))

    pltpu.sync_copy(bnd_hbm.at[row, pl.ds(base, HALF)], bnd_v.at[pl.ds(0, HALF)])

    @pl.when(dirn == 0)
    def _fwd():
        def fbody(l, carry):
            idx = lax.iota(jnp.int32, LANES) + (base + l * LANES)
            b = bnd_v[pl.ds(l * LANES, LANES)]
            flag = (b != 0) | (idx == 0)
            v = jnp.where(flag, idx, -1)
            st = jnp.maximum(plsc.cummax(v), carry)
            res_v[pl.ds(l * LANES, LANES)] = st
            return jnp.max(st)

        fc = lax.fori_loop(0, n_chunks, fbody, jnp.int32(-1))
        pltpu.sync_copy(res_v, start_hbm.at[row, 0, pl.ds(base, HALF)])

        @pl.when(half == 0)
        def _():
            sum_v[...] = jnp.full((LANES,), fc, jnp.int32)
            pltpu.sync_copy(sum_v, fsum_hbm.at[row])

    @pl.when(dirn == 1)
    def _bwd():
        bnd_v[pl.ds(HALF, LANES)] = jnp.zeros((LANES,), jnp.int32)

        def bbody(k, carry):
            l = n_chunks - 1 - k
            off = l * LANES + 1
            u = bnd_v[pl.ds(off, LANES)]
            idx2 = lax.iota(jnp.int32, LANES) + (base + off)
            m = jnp.where(u != 0, idx2, S)
            r = lax.rev(m, (0,))
            ipm = -plsc.cummax(-r)
            esm = lax.rev(ipm, (0,))
            e = jnp.minimum(esm, carry)
            res_v[pl.ds(l * LANES, LANES)] = e
            return jnp.minimum(jnp.min(m), carry)

        bc = lax.fori_loop(0, n_chunks, bbody, jnp.int32(S))
        pltpu.sync_copy(res_v, end_hbm.at[row, 0, pl.ds(base, HALF)])

        @pl.when(half == 1)
        def _():
            b0 = bnd_v[pl.ds(0, LANES)]
            li = lax.iota(jnp.int32, LANES)
            m0 = jnp.where((li == 0) & (b0 != 0), jnp.int32(base), S)
            bs = jnp.minimum(bc, jnp.min(m0))
            sum_v[...] = jnp.full((LANES,), bs, jnp.int32)
            pltpu.sync_copy(sum_v, bsum_hbm.at[row])

    for c in range(NCH):
        handles[c].wait()
        pltpu.sync_copy(rows_v.at[c], cg_hbm.at[gb, pl.ds(gs0 + c * GC, GC)])


def _sc_segments(bnd, orig, ctab):
    B, S = bnd.shape
    HALF = S // 2
    mesh = plsc.VectorSubcoreMesh(core_axis_name="c", subcore_axis_name="s")
    return pl.kernel(
        functools.partial(_sc_body, S, B),
        out_type=[
            jax.ShapeDtypeStruct((B, 1, S), jnp.int32),
            jax.ShapeDtypeStruct((B, 1, S), jnp.int32),
            jax.ShapeDtypeStruct((B, LANES), jnp.int32),
            jax.ShapeDtypeStruct((B, LANES), jnp.int32),
            jax.ShapeDtypeStruct((B, S, 128), jnp.float32),
        ],
        mesh=mesh,
        compiler_params=pltpu.CompilerParams(needs_layout_passes=False,
                                             use_tc_tiling_on_sc=True),
        scratch_types=[
            pltpu.VMEM((HALF + LANES,), jnp.int32),
            pltpu.VMEM((HALF,), jnp.int32),
            pltpu.VMEM((LANES,), jnp.int32),
            pltpu.VMEM((4, 128), jnp.int32),
            pltpu.VMEM((4, 128, 128), jnp.float32),
            pltpu.SemaphoreType.DMA,
        ],
    )(bnd, orig, ctab)


def _assemble_body(S, S_blk, base_pe_ref, wproj_ref, wseg_ref, bseg_ref,
                   bproj_ref, ss_ref, se_ref, op_ref, cg_ref, mfb_ref, pm_ref,
                   fs_ref, bs_ref, out_ref, shared_sc, wc_sc):
    i = pl.program_id(0)
    j = pl.program_id(1)
    HALF = S // 2

    @pl.when(j == 0)
    def _():
        W1 = wproj_ref[:, :D_MODEL]
        W2 = wproj_ref[:, D_MODEL:]
        sh0 = lax.dot_general(base_pe_ref[...], W1, (((1,), (1,)), ((), ())),
                              preferred_element_type=jnp.float32)
        sb = lax.dot_general(bseg_ref[...], W2, (((1,), (1,)), ((), ())),
                             preferred_element_type=jnp.float32)
        shared_sc[...] = sh0 + bproj_ref[...] + sb

    @pl.when((i == 0) & (j == 0))
    def _():
        W2 = wproj_ref[:, D_MODEL:]
        wc_sc[...] = lax.dot_general(wseg_ref[...], W2, (((0,), (1,)), ((), ())),
                                     preferred_element_type=jnp.float32)

    sh = shared_sc[...]
    s_loc = ss_ref[0]
    e_loc = se_ref[0]
    pos_row = op_ref[0]
    fs = jnp.max(fs_ref[0])
    bs = jnp.min(bs_ref[0])
    first = i < (HALF // S_blk)
    s = jnp.maximum(s_loc, jnp.where(first, -1, fs))
    e = jnp.minimum(e_loc, jnp.where(first, bs, S))
    tok = lax.broadcasted_iota(jnp.int32, (1, S_blk), 1) + i * S_blk
    ln = (e - s).astype(jnp.float32)
    pos_in = (tok - s).astype(jnp.float32) / jnp.maximum(ln - 1.0, 1.0)
    len_norm = ln * (1.0 / float(S))
    gpos = pos_row.astype(jnp.float32) * (1.0 / float(MAX_LEN))
    fT = jnp.concatenate([pos_in, len_norm, gpos], axis=0)
    seg_add = lax.dot_general(fT, wc_sc[...], (((0,), (0,)), ((), ())),
                              preferred_element_type=jnp.float32)
    r = sh + seg_add
    posf_row = op_ref[0].astype(jnp.float32)
    motif_t = jnp.sin(mfb_ref[...] * posf_row)
    mg_add = lax.dot_general(motif_t, pm_ref[...], (((0,), (0,)), ((), ())),
                             preferred_element_type=jnp.float32)
    full = jnp.concatenate([
        r[:, 0:128] + cg_ref[0],
        r[:, 128:896],
        r[:, 896:1024] + mg_add,
    ], axis=1)
    out_ref[...] = full[None]


def _assemble_call(base_pe, W_proj, W_seg, b_seg2, b_proj2,
                   ss2, se2, op2, cg, mf, pmat, fsum3, bsum3):
    B = ss2.shape[0]
    S = base_pe.shape[0]
    S_blk = 1024
    n_s = S // S_blk
    mfb = jnp.broadcast_to(mf[:, None], (16, S_blk))
    row_spec = pl.BlockSpec((1, 1, S_blk), lambda i, j: (j, 0, i))
    return pl.pallas_call(
        functools.partial(_assemble_body, S, S_blk),
        grid=(n_s, B),
        in_specs=[
            pl.BlockSpec((S_blk, D_MODEL), lambda i, j: (i, 0)),
            pl.BlockSpec((D_MODEL, D_MODEL + SEG_DIM), lambda i, j: (0, 0)),
            pl.BlockSpec((SEG_DIM, 3), lambda i, j: (0, 0)),
            pl.BlockSpec((1, SEG_DIM), lambda i, j: (0, 0)),
            pl.BlockSpec((1, D_MODEL), lambda i, j: (0, 0)),
            row_spec, row_spec, row_spec,
            pl.BlockSpec((1, S_blk, 128), lambda i, j: (j, i, 0)),
            pl.BlockSpec((16, S_blk), lambda i, j: (0, 0)),
            pl.BlockSpec((16, 128), lambda i, j: (0, 0)),
            pl.BlockSpec((1, 1, LANES), lambda i, j: (j, 0, 0)),
            pl.BlockSpec((1, 1, LANES), lambda i, j: (j, 0, 0)),
        ],
        out_specs=pl.BlockSpec((1, S_blk, D_MODEL), lambda i, j: (j, i, 0)),
        out_shape=jax.ShapeDtypeStruct((B, S, D_MODEL), jnp.float32),
        scratch_shapes=[
            pltpu.VMEM((S_blk, D_MODEL), jnp.float32),
            pltpu.VMEM((3, D_MODEL), jnp.float32),
        ],
    )(base_pe, W_proj, W_seg, b_seg2, b_proj2,
      ss2, se2, op2, cg, mfb, pmat, fsum3, bsum3)


@functools.lru_cache(maxsize=2)
def _base_pe_table(seq_len, d_model):
    pos = np.arange(seq_len, dtype=np.float32)[:, None]
    div = np.exp(np.arange(0, d_model, 2, dtype=np.float32)
                 * (-np.log(10000.0) / d_model)).astype(np.float32)
    pe = np.zeros((seq_len, d_model), dtype=np.float32)
    pe[:, 0::2] = np.sin((pos * div).astype(np.float32))
    pe[:, 1::2] = np.cos((pos * div).astype(np.float32))
    return jnp.asarray(pe)


@functools.lru_cache(maxsize=1)
def _sin_tables():
    cf = np.arange(0, D_MODEL // 4, 2, dtype=np.float32) * (2.0 * np.pi / 3.0)
    cf = cf.astype(np.float32)
    parts = []
    for period in [8, 10, 21, 147]:
        parts.append(np.arange(0, min(D_MODEL // 8, 8), 2, dtype=np.float32)
                     * (2.0 * np.pi / period))
    mf = np.concatenate(parts).astype(np.float32)
    p = np.arange(MAX_LEN, dtype=np.float32)[:, None]
    cargs = (p * cf[None, :]).astype(np.float32)
    ctab = (np.float32(0.1)
            * np.sin(cargs.astype(np.float64)).astype(np.float32))
    ctab = jnp.asarray(ctab)
    pmat = np.zeros((16, 128), dtype=np.float32)
    pmat[np.arange(16), 112 + np.arange(16)] = 0.05
    return ctab, jnp.asarray(mf), jnp.asarray(pmat)


def kernel(seq_len, patch_boundaries, original_positions,
           W_seg, b_seg, W_proj, b_proj):
    B, S = patch_boundaries.shape
    bnd = patch_boundaries.astype(jnp.int32)
    orig = original_positions.astype(jnp.int32)

    ctab, mf, pmat = _sin_tables()
    seg_start, seg_end, fsum, bsum, cg = _sc_segments(bnd, orig, ctab)

    base_pe = _base_pe_table(S, D_MODEL)
    out = _assemble_call(
        base_pe, W_proj, W_seg,
        b_seg.reshape(1, SEG_DIM), b_proj.reshape(1, D_MODEL),
        seg_start, seg_end,
        orig.reshape(B, 1, S), cg, mf, pmat,
        fsum.reshape(B, 1, LANES), bsum.reshape(B, 1, LANES))
    return out
